# baseline (speedup 1.0000x reference)
import functools

import jax
import jax.numpy as jnp
from jax import lax
from jax.experimental import pallas as pl
from jax.experimental.pallas import tpu as pltpu
from jax.experimental.pallas import tpu_sc as plsc

BATCH = 8192
FEAT = 2048
LANES = 16
NUM_CORES = 2
NUM_SUBCORES = 16
NW = NUM_CORES * NUM_SUBCORES
R = 8
BLK = R * FEAT
NGROUPS = BATCH // R // NW
UNROLL = 4
NBUF = 3


def _make_permute():
    mesh = plsc.VectorSubcoreMesh(core_axis_name="c", subcore_axis_name="s")

    @functools.partial(
        pl.kernel,
        out_type=jax.ShapeDtypeStruct((BATCH, FEAT), jnp.float32),
        mesh=mesh,
        compiler_params=pltpu.CompilerParams(needs_layout_passes=False),
        scratch_types=[
            pltpu.VMEM((FEAT,), jnp.int32),
            pltpu.VMEM((R, FEAT), jnp.float32),
            pltpu.VMEM((R, FEAT), jnp.float32),
            pltpu.VMEM((R, FEAT), jnp.float32),
            pltpu.VMEM((R, FEAT), jnp.float32),
            pltpu.VMEM((R, FEAT), jnp.float32),
            pltpu.VMEM((R, FEAT), jnp.float32),
            pltpu.SemaphoreType.DMA,
            pltpu.SemaphoreType.DMA,
            pltpu.SemaphoreType.DMA,
            pltpu.SemaphoreType.DMA,
            pltpu.SemaphoreType.DMA,
            pltpu.SemaphoreType.DMA,
        ],
    )
    def permute(in_hbm, perm_hbm, out_hbm, perm_v, in0, in1, in2,
                out0, out1, out2, si0, si1, si2, so0, so1, so2):
        wid = lax.axis_index("s") * NUM_CORES + lax.axis_index("c")
        base = wid * NGROUPS
        ins = (in0, in1, in2)
        outs = (out0, out1, out2)
        sin = (si0, si1, si2)
        sout = (so0, so1, so2)

        pltpu.sync_copy(perm_hbm, perm_v)

        def start_in(g, b):
            pltpu.async_copy(
                in_hbm.at[pl.ds((base + g) * R, R), :], ins[b], sin[b])

        def wait_in(g, b):
            pltpu.make_async_copy(
                in_hbm.at[pl.ds((base + g) * R, R), :], ins[b],
                sin[b]).wait()

        def start_out(g, b):
            pltpu.async_copy(
                outs[b], out_hbm.at[pl.ds((base + g) * R, R), :], sout[b])

        HALF = FEAT // 2

        def start_out_half(g, b, h):
            pltpu.async_copy(
                outs[b].at[:, pl.ds(h * HALF, HALF)],
                out_hbm.at[pl.ds((base + g) * R, R), pl.ds(h * HALF, HALF)],
                sout[b])

        def wait_out(g, b):
            pltpu.make_async_copy(
                outs[b], out_hbm.at[pl.ds((base + g) * R, R), :],
                sout[b]).wait()

        for b in range(NBUF):
            start_in(b, b)

        def do_group(g, b):
            wait_in(g, b)
            @pl.when(g >= NBUF)
            def _():
                wait_out(g - NBUF, b)

            in_b = ins[b]
            out_b = outs[b]

            for h in range(2):
                @plsc.parallel_loop(h * (FEAT // LANES // 2),
                                    (h + 1) * (FEAT // LANES // 2),
                                    unroll=UNROLL)
                def jbody(j):
                    j16 = j * LANES
                    idx = perm_v[pl.ds(j16, LANES)]
                    for r in range(R):
                        rv = jnp.full((LANES,), r, jnp.int32)
                        vals = plsc.load_gather(in_b, [rv, idx])
                        out_b[r, pl.ds(j16, LANES)] = vals

                start_out_half(g, b, h)

            @pl.when(g + NBUF < NGROUPS)
            def _():
                start_in(g + NBUF, b)

        def trip(h, carry):
            for b in range(NBUF):
                do_group(NBUF * h + b, b)
            return carry

        nfull = NGROUPS // NBUF
        lax.fori_loop(0, nfull, trip, 0)
        for g in range(nfull * NBUF, NGROUPS):
            do_group(g, g % NBUF)
        for g in range(NGROUPS - NBUF, NGROUPS):
            wait_out(g, g % NBUF)

    return permute


_permute = _make_permute()


def kernel(inputs, perm):
    out = _permute(inputs, perm)
    logdet = jnp.zeros((inputs.shape[0], 1), dtype=inputs.dtype)
    return (out, logdet)


# speedup vs baseline: 1.0167x; 1.0167x over previous
import functools

import jax
import jax.numpy as jnp
from jax import lax
from jax.experimental import pallas as pl
from jax.experimental.pallas import tpu as pltpu
from jax.experimental.pallas import tpu_sc as plsc

BATCH = 8192
FEAT = 2048
LANES = 16
NUM_CORES = 2
NUM_SUBCORES = 16
NW = NUM_CORES * NUM_SUBCORES
R = 8
BLK = R * FEAT
NGROUPS = BATCH // R // NW
UNROLL = 4
NBUF = 3


def _make_permute():
    mesh = plsc.VectorSubcoreMesh(core_axis_name="c", subcore_axis_name="s")

    @functools.partial(
        pl.kernel,
        out_type=jax.ShapeDtypeStruct((BATCH, FEAT), jnp.float32),
        mesh=mesh,
        compiler_params=pltpu.CompilerParams(needs_layout_passes=False),
        scratch_types=[
            pltpu.VMEM((FEAT,), jnp.int32),
            pltpu.VMEM((R, FEAT), jnp.float32),
            pltpu.VMEM((R, FEAT), jnp.float32),
            pltpu.VMEM((R, FEAT), jnp.float32),
            pltpu.VMEM((R, FEAT), jnp.float32),
            pltpu.VMEM((R, FEAT), jnp.float32),
            pltpu.VMEM((R, FEAT), jnp.float32),
            pltpu.SemaphoreType.DMA,
            pltpu.SemaphoreType.DMA,
            pltpu.SemaphoreType.DMA,
            pltpu.SemaphoreType.DMA,
            pltpu.SemaphoreType.DMA,
            pltpu.SemaphoreType.DMA,
        ],
    )
    def permute(in_hbm, perm_hbm, out_hbm, perm_v, in0, in1, in2,
                out0, out1, out2, si0, si1, si2, so0, so1, so2):
        wid = lax.axis_index("s") * NUM_CORES + lax.axis_index("c")
        base = wid * NGROUPS
        ins = (in0, in1, in2)
        outs = (out0, out1, out2)
        sin = (si0, si1, si2)
        sout = (so0, so1, so2)

        pltpu.sync_copy(perm_hbm, perm_v)

        def start_in(g, b):
            pltpu.async_copy(
                in_hbm.at[pl.ds((base + g) * R, R), :], ins[b], sin[b])

        def wait_in(g, b):
            pltpu.make_async_copy(
                in_hbm.at[pl.ds((base + g) * R, R), :], ins[b],
                sin[b]).wait()

        def start_out(g, b):
            pltpu.async_copy(
                outs[b], out_hbm.at[pl.ds((base + g) * R, R), :], sout[b])

        def wait_out(g, b):
            pltpu.make_async_copy(
                outs[b], out_hbm.at[pl.ds((base + g) * R, R), :],
                sout[b]).wait()

        for b in range(NBUF):
            start_in(b, b)

        def do_group(g, b):
            wait_in(g, b)
            @pl.when(g >= NBUF)
            def _():
                wait_out(g - NBUF, b)

            in_b = ins[b]
            out_b = outs[b]

            @plsc.parallel_loop(0, FEAT // LANES, unroll=UNROLL)
            def jbody(j):
                j16 = j * LANES
                idx = perm_v[pl.ds(j16, LANES)]
                for r in range(R):
                    rv = jnp.full((LANES,), r, jnp.int32)
                    vals = plsc.load_gather(in_b, [rv, idx])
                    out_b[r, pl.ds(j16, LANES)] = vals

            start_out(g, b)

            @pl.when(g + NBUF < NGROUPS)
            def _():
                start_in(g + NBUF, b)

        def trip(h, carry):
            for b in range(NBUF):
                do_group(NBUF * h + b, b)
            return carry

        nfull = NGROUPS // NBUF
        lax.fori_loop(0, nfull, trip, 0)
        for g in range(nfull * NBUF, NGROUPS):
            do_group(g, g % NBUF)
        for g in range(NGROUPS - NBUF, NGROUPS):
            wait_out(g, g % NBUF)

    return permute


_permute = _make_permute()


def kernel(inputs, perm):
    out = _permute(inputs, perm)
    logdet = jnp.zeros((inputs.shape[0], 1), dtype=inputs.dtype)
    return (out, logdet)
